# Initial kernel scaffold; baseline (speedup 1.0000x reference)
import functools

import jax
import jax.numpy as jnp
from jax import lax
from jax.experimental import pallas as pl
from jax.experimental.pallas import tpu as pltpu
from jax.experimental.pallas import tpu_sc as plsc

_NC = 2
_NS = 16
_NW = _NC * _NS
_LANES = 16


@functools.lru_cache(maxsize=None)
def _make_sc_scores(vocab: int, dim: int, batch: int, nneg: int):
    assert dim == 64 and batch % (_NW * 128) == 0
    b_per_w = batch // _NW
    nblk = b_per_w // 128

    mesh = plsc.VectorSubcoreMesh(core_axis_name="c", subcore_axis_name="s")

    @functools.partial(
        pl.kernel,
        mesh=mesh,
        out_type=[
            jax.ShapeDtypeStruct((batch,), jnp.float32),
            jax.ShapeDtypeStruct((batch,), jnp.float32),
        ],
        scratch_types=[
            pltpu.VMEM((b_per_w,), jnp.int32),
            pltpu.VMEM((b_per_w,), jnp.int32),
            pltpu.VMEM((nneg, b_per_w), jnp.int32),
            pltpu.VMEM((128, 64), jnp.float32),
            pltpu.VMEM((128, 64), jnp.float32),
            pltpu.VMEM((128, 64), jnp.float32),
            pltpu.VMEM((128, 64), jnp.float32),
            pltpu.VMEM((128, 64), jnp.float32),
            pltpu.VMEM((b_per_w,), jnp.float32),
            pltpu.VMEM((b_per_w,), jnp.float32),
            pltpu.SemaphoreType.DMA,
            pltpu.SemaphoreType.DMA,
            pltpu.SemaphoreType.DMA,
            pltpu.SemaphoreType.DMA,
            pltpu.SemaphoreType.DMA,
        ],
    )
    def sc_scores(up_hbm, vp_hbm, nt_hbm, uw_hbm, vw_hbm, sc_hbm, nsc_hbm,
                  uidx, vidx, nidx, eu, ev, ns, bufa, bufb, scv, nscv,
                  sem_eu, sem_ev, sem_ns, sem_a, sem_b):
        wid = lax.axis_index("s") * _NC + lax.axis_index("c")
        base = wid * b_per_w
        pltpu.sync_copy(up_hbm.at[pl.ds(base, b_per_w)], uidx)
        pltpu.sync_copy(vp_hbm.at[pl.ds(base, b_per_w)], vidx)
        for n in range(nneg):
            pltpu.sync_copy(nt_hbm.at[n, pl.ds(base, b_per_w)], nidx.at[n])

        bufs = [bufa, bufb]
        sems = [sem_a, sem_b]
        for blk in range(nblk):
            off = blk * 128
            cp_eu = pltpu.async_copy(uw_hbm.at[uidx.at[pl.ds(off, 128)]], eu, sem_eu)
            cp_ev = pltpu.async_copy(vw_hbm.at[vidx.at[pl.ds(off, 128)]], ev, sem_ev)
            cp_ns = pltpu.async_copy(vw_hbm.at[nidx.at[0, pl.ds(off, 128)]], ns, sem_ns)
            pending = {1: pltpu.async_copy(
                vw_hbm.at[nidx.at[1, pl.ds(off, 128)]], bufs[1], sems[1])}
            cp_ns.wait()
            for n in range(1, nneg):
                if n + 1 < nneg:
                    pending[n + 1] = pltpu.async_copy(
                        vw_hbm.at[nidx.at[n + 1, pl.ds(off, 128)]],
                        bufs[(n + 1) % 2], sems[(n + 1) % 2])
                pending[n].wait()
                buf = bufs[n % 2]

                def acc_body(r, carry, buf=buf):
                    for c in range(64 // _LANES):
                        plsc.addupdate(ns.at[r, pl.ds(c * _LANES, _LANES)],
                                       buf[r, pl.ds(c * _LANES, _LANES)])
                    return carry

                lax.fori_loop(0, 128, acc_body, 0)

            cp_eu.wait()
            cp_ev.wait()
            iota16 = lax.iota(jnp.int32, _LANES)
            for rb in range(128 // _LANES):
                rows = iota16 + (rb * _LANES)
                zf = jnp.zeros((_LANES,), jnp.float32)
                col0 = jnp.zeros((_LANES,), jnp.int32)

                def d_body(d, carry):
                    sacc, nacc, col = carry
                    e = plsc.load_gather(eu, [rows, col])
                    v = plsc.load_gather(ev, [rows, col])
                    g = plsc.load_gather(ns, [rows, col])
                    return (sacc + e * v, nacc + e * g, col + 1)

                sacc, nacc, _ = lax.fori_loop(0, 64, d_body, (zf, zf, col0))
                scv[pl.ds(off + rb * _LANES, _LANES)] = sacc
                nscv[pl.ds(off + rb * _LANES, _LANES)] = nacc

        pltpu.sync_copy(scv, sc_hbm.at[pl.ds(base, b_per_w)])
        pltpu.sync_copy(nscv, nsc_hbm.at[pl.ds(base, b_per_w)])

    return sc_scores


def _tc_loss_body(s_ref, n_ref, o_ref):
    s = s_ref[...]
    t = n_ref[...]
    ls = jnp.minimum(s, 0.0) - jnp.log(1.0 + jnp.exp(-jnp.abs(s)))
    ln = jnp.minimum(-t, 0.0) - jnp.log(1.0 + jnp.exp(-jnp.abs(t)))
    total = -jnp.sum(ls + ln) / (s.shape[0] * s.shape[1])
    o_ref[...] = jnp.reshape(total, (1, 1))


def kernel(u_pos, v_pos, v_neg, batch_size, u_weight, v_weight):
    batch = u_pos.shape[0]
    vocab, dim = u_weight.shape
    nneg = v_neg.shape[1]

    u_idx = u_pos.astype(jnp.int32)
    v_idx = v_pos.astype(jnp.int32)
    neg_t = v_neg.astype(jnp.int32).T

    sc_fn = _make_sc_scores(vocab, dim, batch, nneg)
    scores, nscores = sc_fn(u_idx, v_idx, neg_t, u_weight, v_weight)

    rows = batch // 128
    loss = pl.pallas_call(
        _tc_loss_body,
        out_shape=jax.ShapeDtypeStruct((1, 1), jnp.float32),
    )(scores.reshape(rows, 128), nscores.reshape(rows, 128))
    return loss[0, 0]


# speedup vs baseline: 5.1676x; 5.1676x over previous
import functools

import jax
import jax.numpy as jnp
from jax import lax
from jax.experimental import pallas as pl
from jax.experimental.pallas import tpu as pltpu
from jax.experimental.pallas import tpu_sc as plsc

_NC = 2
_NS = 16
_NW = _NC * _NS
_LANES = 16


@functools.lru_cache(maxsize=None)
def _make_sc_gather(vocab: int, dim: int, batch: int, nneg: int):
    assert dim == 64 and batch % (_NW * 128) == 0
    b_per_w = batch // _NW
    nblk = b_per_w // 128

    mesh = plsc.VectorSubcoreMesh(core_axis_name="c", subcore_axis_name="s")

    @functools.partial(
        pl.kernel,
        mesh=mesh,
        compiler_params=pltpu.CompilerParams(use_tc_tiling_on_sc=False),
        out_type=[
            jax.ShapeDtypeStruct((batch, dim), jnp.float32),
            jax.ShapeDtypeStruct((batch, dim), jnp.float32),
            jax.ShapeDtypeStruct((batch, dim), jnp.float32),
        ],
        scratch_types=[
            pltpu.VMEM((b_per_w,), jnp.int32),
            pltpu.VMEM((b_per_w,), jnp.int32),
            pltpu.VMEM((nneg, b_per_w), jnp.int32),
            pltpu.VMEM((128, 64), jnp.float32),
            pltpu.VMEM((128, 64), jnp.float32),
            pltpu.VMEM((128, 64), jnp.float32),
            pltpu.VMEM((128, 64), jnp.float32),
            pltpu.VMEM((128, 64), jnp.float32),
            pltpu.SemaphoreType.DMA,
            pltpu.SemaphoreType.DMA,
            pltpu.SemaphoreType.DMA,
            pltpu.SemaphoreType.DMA,
            pltpu.SemaphoreType.DMA,
        ],
    )
    def sc_gather(up_hbm, vp_hbm, nt_hbm, uw_hbm, vw_hbm,
                  eu_hbm, ev_hbm, ns_hbm,
                  uidx, vidx, nidx, eu, ev, ns, bufa, bufb,
                  sem_eu, sem_ev, sem_ns, sem_a, sem_b):
        wid = lax.axis_index("s") * _NC + lax.axis_index("c")
        base = wid * b_per_w
        pltpu.sync_copy(up_hbm.at[pl.ds(base, b_per_w)], uidx)
        pltpu.sync_copy(vp_hbm.at[pl.ds(base, b_per_w)], vidx)
        for n in range(nneg):
            pltpu.sync_copy(nt_hbm.at[n, pl.ds(base, b_per_w)], nidx.at[n])

        bufs = [bufa, bufb]
        sems = [sem_a, sem_b]
        for blk in range(nblk):
            off = blk * 128
            cp_eu = pltpu.async_copy(uw_hbm.at[uidx.at[pl.ds(off, 128)]], eu, sem_eu)
            cp_ev = pltpu.async_copy(vw_hbm.at[vidx.at[pl.ds(off, 128)]], ev, sem_ev)
            cp_ns = pltpu.async_copy(vw_hbm.at[nidx.at[0, pl.ds(off, 128)]], ns, sem_ns)
            pending = {1: pltpu.async_copy(
                vw_hbm.at[nidx.at[1, pl.ds(off, 128)]], bufs[1], sems[1])}
            cp_ns.wait()
            for n in range(1, nneg):
                if n + 1 < nneg:
                    pending[n + 1] = pltpu.async_copy(
                        vw_hbm.at[nidx.at[n + 1, pl.ds(off, 128)]],
                        bufs[(n + 1) % 2], sems[(n + 1) % 2])
                pending[n].wait()
                buf = bufs[n % 2]

                def acc_body(r, carry, buf=buf):
                    for c in range(64 // _LANES):
                        plsc.addupdate(ns.at[r, pl.ds(c * _LANES, _LANES)],
                                       buf[r, pl.ds(c * _LANES, _LANES)])
                    return carry

                lax.fori_loop(0, 128, acc_body, 0)

            cp_eu.wait()
            cp_ev.wait()
            pltpu.sync_copy(eu, eu_hbm.at[pl.ds(base + off, 128)])
            pltpu.sync_copy(ev, ev_hbm.at[pl.ds(base + off, 128)])
            pltpu.sync_copy(ns, ns_hbm.at[pl.ds(base + off, 128)])

    return sc_gather


def _tc_loss_body(eu_ref, ev_ref, ns_ref, o_ref):
    i = pl.program_id(0)
    eu = eu_ref[...]
    s = jnp.sum(eu * ev_ref[...], axis=1)
    t = jnp.sum(eu * ns_ref[...], axis=1)
    ls = jnp.minimum(s, 0.0) - jnp.log(1.0 + jnp.exp(-jnp.abs(s)))
    ln = jnp.minimum(-t, 0.0) - jnp.log(1.0 + jnp.exp(-jnp.abs(t)))
    partial = -jnp.sum(ls + ln)

    @pl.when(i == 0)
    def _():
        o_ref[...] = jnp.zeros_like(o_ref)

    o_ref[...] += jnp.reshape(partial, (1, 1))


def kernel(u_pos, v_pos, v_neg, batch_size, u_weight, v_weight):
    batch = u_pos.shape[0]
    vocab, dim = u_weight.shape
    nneg = v_neg.shape[1]

    u_idx = u_pos.astype(jnp.int32)
    v_idx = v_pos.astype(jnp.int32)
    neg_t = v_neg.astype(jnp.int32).T

    sc_fn = _make_sc_gather(vocab, dim, batch, nneg)
    eu_all, ev_all, ns_all = sc_fn(u_idx, v_idx, neg_t, u_weight, v_weight)

    rows_per_blk = 2048
    grid = batch // rows_per_blk
    spec = pl.BlockSpec((rows_per_blk, dim), lambda i: (i, 0))
    total = pl.pallas_call(
        _tc_loss_body,
        grid=(grid,),
        in_specs=[spec, spec, spec],
        out_specs=pl.BlockSpec((1, 1), lambda i: (0, 0)),
        out_shape=jax.ShapeDtypeStruct((1, 1), jnp.float32),
    )(eu_all, ev_all, ns_all)
    return total[0, 0] / batch
